# baseline (speedup 1.0000x reference)
import jax
import jax.numpy as jnp
from jax import lax
from jax.experimental import pallas as pl
from jax.experimental.pallas import tpu as pltpu
from jax.experimental.pallas import tpu_sc as plsc

PERIODS = 288
WEEKEND = 7
P_DIM = 24
W_DIM = 24
OUT_DIM = P_DIM + W_DIM
PAD_DIM = 128
N_TAB = PERIODS * WEEKEND
N_ROWS = 64 * 12 * 1000
NW = 32
PER_W = N_ROWS // NW
SUB = 48
NSUB = 10
B_BLK = SUB * NSUB
N_BLK = PER_W // B_BLK
GROUPS = SUB // 16
DEPTH = 5


def _build_table_kernel(p_ref, w_ref, o_ref):
    pe = jnp.broadcast_to(p_ref[:][:, None, :], (PERIODS, WEEKEND, P_DIM))
    we = jnp.broadcast_to(w_ref[:][None, :, :], (PERIODS, WEEKEND, W_DIM))
    o_ref[:] = jnp.concatenate([pe, we], axis=-1)


def _build_fused_table(periods_embedding, weekend_embedding):
    fused3 = pl.pallas_call(
        _build_table_kernel,
        out_shape=jax.ShapeDtypeStruct((PERIODS, WEEKEND, OUT_DIM), jnp.float32),
    )(periods_embedding, weekend_embedding)
    return fused3.reshape(N_TAB, OUT_DIM)


def _sc_body(xp_hbm, xw_hbm, tab_hbm, out_hbm, xp_v, xw_v, idx_v, rows_v, xs, gs, osem):
    wid = lax.axis_index("s") * 2 + lax.axis_index("c")
    w_base = wid * PER_W

    def fire_x(i, s):
        base = w_base + i * B_BLK
        pltpu.async_copy(xp_hbm.at[pl.ds(base, B_BLK)], xp_v[s], xs[s])
        pltpu.async_copy(xw_hbm.at[pl.ds(base, B_BLK)], xw_v[s], xs[s])

    def wait_x(i, s):
        base = w_base + i * B_BLK
        pltpu.make_async_copy(xp_hbm.at[pl.ds(base, B_BLK)], xp_v[s], xs[s]).wait()
        pltpu.make_async_copy(xw_hbm.at[pl.ds(base, B_BLK)], xw_v[s], xs[s]).wait()

    def compute_idx(s):
        for r in range(NSUB):
            for g in range(GROUPS):
                o = r * SUB + g * 16
                pv = xp_v[s][pl.ds(o, 16)]
                wv = xw_v[s][pl.ds(o, 16)]
                pi = jnp.minimum((pv * float(PERIODS)).astype(jnp.int32), PERIODS - 1)
                wi = jnp.minimum(wv.astype(jnp.int32), WEEKEND - 1)
                idx_v[s][r, pl.ds(g * 16, 16)] = pi * WEEKEND + wi

    def fire_gathers(s):
        for r in range(NSUB):
            pltpu.async_copy(
                tab_hbm.at[idx_v[s].at[r]], rows_v[s].at[pl.ds(r * SUB, SUB)], gs[s]
            )

    def wait_gathers(s):
        for r in range(NSUB):
            pltpu.make_async_copy(
                tab_hbm.at[idx_v[s].at[r]], rows_v[s].at[pl.ds(r * SUB, SUB)], gs[s]
            ).wait()

    def fire_out(i, s):
        base = w_base + i * B_BLK
        pltpu.async_copy(
            rows_v[s], out_hbm.at[pl.ds(base, B_BLK), pl.ds(0, OUT_DIM)], osem[s]
        )

    def wait_out(i, s):
        base = w_base + i * B_BLK
        pltpu.make_async_copy(
            rows_v[s], out_hbm.at[pl.ds(base, B_BLK), pl.ds(0, OUT_DIM)], osem[s]
        ).wait()

    for s in range(DEPTH):
        fire_x(s, s)
    for i in range(DEPTH):
        wait_x(i, i)
        compute_idx(i)
        fire_x(i + DEPTH, i)
        fire_gathers(i)
        if i >= 2:
            wait_gathers(i - 2)
            fire_out(i - 2, i - 2)

    def steady(j, carry):
        i0 = DEPTH * j
        for d in range(DEPTH):
            i = i0 + d
            s = d
            wait_x(i, s)
            compute_idx(s)

            @pl.when(i + DEPTH < N_BLK)
            def _():
                fire_x(i + DEPTH, s)

            wait_out(i - DEPTH, s)
            fire_gathers(s)
            wait_gathers((s - 2) % DEPTH)
            fire_out(i - 2, (s - 2) % DEPTH)
        return carry

    lax.fori_loop(1, N_BLK // DEPTH, steady, 0)

    for i in (N_BLK - 2, N_BLK - 1):
        wait_gathers(i % DEPTH)
        fire_out(i, i % DEPTH)
    for s in range(DEPTH):
        wait_out(N_BLK - DEPTH + s, s)


@jax.jit
def _encode(xp, xw, fused_table):
    mesh = plsc.VectorSubcoreMesh(core_axis_name="c", subcore_axis_name="s")
    return pl.kernel(
        _sc_body,
        out_type=jax.ShapeDtypeStruct((N_ROWS, PAD_DIM), jnp.float32),
        mesh=mesh,
        compiler_params=pltpu.CompilerParams(
            needs_layout_passes=False, use_tc_tiling_on_sc=False
        ),
        scratch_types=dict(
            xp_v=[pltpu.VMEM((B_BLK,), jnp.float32) for _ in range(DEPTH)],
            xw_v=[pltpu.VMEM((B_BLK,), jnp.float32) for _ in range(DEPTH)],
            idx_v=[pltpu.VMEM((NSUB, SUB), jnp.int32) for _ in range(DEPTH)],
            rows_v=[pltpu.VMEM((B_BLK, OUT_DIM), jnp.float32) for _ in range(DEPTH)],
            xs=[pltpu.SemaphoreType.DMA for _ in range(DEPTH)],
            gs=[pltpu.SemaphoreType.DMA for _ in range(DEPTH)],
            osem=[pltpu.SemaphoreType.DMA for _ in range(DEPTH)],
        ),
    )(xp, xw, fused_table)


def kernel(x, periods_embedding, weekend_embedding):
    b, t, n, _ = x.shape
    fused = _build_fused_table(periods_embedding, weekend_embedding)
    xp = x[..., 1].reshape(-1)
    xw = x[..., 2].reshape(-1)
    out = _encode(xp, xw, fused)
    return out[:, :OUT_DIM].reshape(b, t, n, OUT_DIM)


# speedup vs baseline: 1.8696x; 1.8696x over previous
import jax
import jax.numpy as jnp
from jax import lax
from jax.experimental import pallas as pl
from jax.experimental.pallas import tpu as pltpu
from jax.experimental.pallas import tpu_sc as plsc

PERIODS = 288
WEEKEND = 7
P_DIM = 24
W_DIM = 24
OUT_DIM = P_DIM + W_DIM
PAD_DIM = 128
N_TAB = PERIODS * WEEKEND
N_ROWS = 64 * 12 * 1000
NW = 32
PER_W = N_ROWS // NW
SUB = 96
NSUB = 5
B_BLK = SUB * NSUB
N_BLK = PER_W // B_BLK
GROUPS = SUB // 16
DEPTH = 5


def _build_table_kernel(p_ref, w_ref, o_ref):
    pe = jnp.broadcast_to(p_ref[:][:, None, :], (PERIODS, WEEKEND, P_DIM))
    we = jnp.broadcast_to(w_ref[:][None, :, :], (PERIODS, WEEKEND, W_DIM))
    o_ref[:] = jnp.concatenate([pe, we], axis=-1)


def _build_fused_table(periods_embedding, weekend_embedding):
    fused3 = pl.pallas_call(
        _build_table_kernel,
        out_shape=jax.ShapeDtypeStruct((PERIODS, WEEKEND, OUT_DIM), jnp.float32),
    )(periods_embedding, weekend_embedding)
    return fused3.reshape(N_TAB, OUT_DIM)


def _sc_body(xp_hbm, xw_hbm, tab_hbm, out_hbm, tab_sh, xp_v, xw_v, idx_v, rows_v, xs, gs, osem):
    sid = lax.axis_index("s")
    wid = sid * 2 + lax.axis_index("c")
    w_base = wid * PER_W

    @pl.when(sid == 0)
    def _():
        pltpu.sync_copy(tab_hbm, tab_sh)

    plsc.subcore_barrier()

    def fire_x(i, s):
        base = w_base + i * B_BLK
        pltpu.async_copy(xp_hbm.at[pl.ds(base, B_BLK)], xp_v[s], xs[s])
        pltpu.async_copy(xw_hbm.at[pl.ds(base, B_BLK)], xw_v[s], xs[s])

    def wait_x(i, s):
        base = w_base + i * B_BLK
        pltpu.make_async_copy(xp_hbm.at[pl.ds(base, B_BLK)], xp_v[s], xs[s]).wait()
        pltpu.make_async_copy(xw_hbm.at[pl.ds(base, B_BLK)], xw_v[s], xs[s]).wait()

    def compute_idx(s):
        for r in range(NSUB):
            for g in range(GROUPS):
                o = r * SUB + g * 16
                pv = xp_v[s][pl.ds(o, 16)]
                wv = xw_v[s][pl.ds(o, 16)]
                pi = jnp.minimum((pv * float(PERIODS)).astype(jnp.int32), PERIODS - 1)
                wi = jnp.minimum(wv.astype(jnp.int32), WEEKEND - 1)
                idx_v[s][r, pl.ds(g * 16, 16)] = pi * WEEKEND + wi

    def fire_gathers(s):
        for r in range(NSUB):
            pltpu.async_copy(
                tab_sh.at[idx_v[s].at[r]], rows_v[s].at[pl.ds(r * SUB, SUB)], gs[s]
            )

    def wait_gathers(s):
        for r in range(NSUB):
            pltpu.make_async_copy(
                tab_sh.at[idx_v[s].at[r]], rows_v[s].at[pl.ds(r * SUB, SUB)], gs[s]
            ).wait()

    def fire_out(i, s):
        base = w_base + i * B_BLK
        pltpu.async_copy(
            rows_v[s], out_hbm.at[pl.ds(base, B_BLK), pl.ds(0, OUT_DIM)], osem[s]
        )

    def wait_out(i, s):
        base = w_base + i * B_BLK
        pltpu.make_async_copy(
            rows_v[s], out_hbm.at[pl.ds(base, B_BLK), pl.ds(0, OUT_DIM)], osem[s]
        ).wait()

    for s in range(DEPTH):
        fire_x(s, s)
    for i in range(DEPTH):
        wait_x(i, i)
        compute_idx(i)
        fire_x(i + DEPTH, i)
        fire_gathers(i)
        if i >= 2:
            wait_gathers(i - 2)
            fire_out(i - 2, i - 2)

    def steady(j, carry):
        i0 = DEPTH * j
        for d in range(DEPTH):
            i = i0 + d
            s = d
            wait_x(i, s)
            compute_idx(s)

            @pl.when(i + DEPTH < N_BLK)
            def _():
                fire_x(i + DEPTH, s)

            wait_out(i - DEPTH, s)
            fire_gathers(s)
            wait_gathers((s - 2) % DEPTH)
            fire_out(i - 2, (s - 2) % DEPTH)
        return carry

    lax.fori_loop(1, N_BLK // DEPTH, steady, 0)

    for i in (N_BLK - 2, N_BLK - 1):
        wait_gathers(i % DEPTH)
        fire_out(i, i % DEPTH)
    for s in range(DEPTH):
        wait_out(N_BLK - DEPTH + s, s)


@jax.jit
def _encode(xp, xw, fused_table):
    mesh = plsc.VectorSubcoreMesh(core_axis_name="c", subcore_axis_name="s")
    return pl.kernel(
        _sc_body,
        out_type=jax.ShapeDtypeStruct((N_ROWS, PAD_DIM), jnp.float32),
        mesh=mesh,
        compiler_params=pltpu.CompilerParams(
            needs_layout_passes=False, use_tc_tiling_on_sc=False
        ),
        scratch_types=dict(
            tab_sh=pltpu.VMEM_SHARED((N_TAB, OUT_DIM), jnp.float32),
            xp_v=[pltpu.VMEM((B_BLK,), jnp.float32) for _ in range(DEPTH)],
            xw_v=[pltpu.VMEM((B_BLK,), jnp.float32) for _ in range(DEPTH)],
            idx_v=[pltpu.VMEM((NSUB, SUB), jnp.int32) for _ in range(DEPTH)],
            rows_v=[pltpu.VMEM((B_BLK, OUT_DIM), jnp.float32) for _ in range(DEPTH)],
            xs=[pltpu.SemaphoreType.DMA for _ in range(DEPTH)],
            gs=[pltpu.SemaphoreType.DMA for _ in range(DEPTH)],
            osem=[pltpu.SemaphoreType.DMA for _ in range(DEPTH)],
        ),
    )(xp, xw, fused_table)


def kernel(x, periods_embedding, weekend_embedding):
    b, t, n, _ = x.shape
    fused = _build_fused_table(periods_embedding, weekend_embedding)
    xp = x[..., 1].reshape(-1)
    xw = x[..., 2].reshape(-1)
    out = _encode(xp, xw, fused)
    return out[:, :OUT_DIM].reshape(b, t, n, OUT_DIM)
